# baseline (speedup 1.0000x reference)
import functools

import jax
import jax.numpy as jnp
from jax import lax
from jax.experimental import pallas as pl
from jax.experimental.pallas import tpu as pltpu
from jax.experimental.pallas import tpu_sc as plsc

_NC = 2
_NS = 16
_NW = _NC * _NS

_CHUNK = 64
_CP = 128
_HB = 16


def _nchw_to_pairs(x, b):
    B, C, H, W = x.shape
    nblk = H // _HB
    hbw = _HB * W

    def body(a_ref, b_ref, out_ref):
        t = a_ref[0].reshape(C, hbw).T
        nxt = b_ref[0, :, 0, 0][None, :]
        out_ref[:, 0, :C] = t
        out_ref[: hbw - 1, 1, :C] = t[1:]
        out_ref[hbw - 1:, 1, :C] = nxt
        out_ref[:, :, C:] = jnp.zeros((hbw, 2, _CP - C), jnp.float32)

    return pl.pallas_call(
        body,
        grid=(nblk,),
        in_specs=[
            pl.BlockSpec((1, C, _HB, W), lambda i, b=b: (b, 0, i, 0)),
            pl.BlockSpec((1, C, _HB, W),
                         lambda i, b=b, m=nblk - 1: (b, 0, jnp.minimum(i + 1, m), 0)),
        ],
        out_specs=pl.BlockSpec((hbw, 2, _CP), lambda i: (i, 0, 0)),
        out_shape=jax.ShapeDtypeStruct((H * W, 2, _CP), jnp.float32),
    )(x, x)


def _rows_to_nchw(rows, acc, b, B, C, H, W):
    def body(rows_ref, *refs):
        out_ref = refs[-1]
        t = rows_ref[0][:, :C].T
        out_ref[0] = t.reshape(C, _HB, W)

    in_specs = [pl.BlockSpec((1, _HB * W, _CP), lambda i: (0, i, 0))]
    args = [rows.reshape(1, H * W, _CP)]
    aliases = {}
    if acc is not None:
        in_specs.append(pl.BlockSpec(memory_space=pl.ANY))
        args.append(acc)
        aliases = {1: 0}

    return pl.pallas_call(
        body,
        grid=(H // _HB,),
        in_specs=in_specs,
        out_specs=pl.BlockSpec((1, C, _HB, W), lambda i, b=b: (b, 0, i, 0)),
        out_shape=jax.ShapeDtypeStruct((B, C, H, W), jnp.float32),
        input_output_aliases=aliases,
    )(*args)


def _make_warp(b, C, H, W):
    HW = H * W
    assert HW % _NW == 0
    span = HW // _NW
    n_chunks = span // _CHUNK
    assert span % _CHUNK == 0 and n_chunks % 2 == 0
    cblocks = C // 16
    assert C % 16 == 0

    mesh = plsc.VectorSubcoreMesh(core_axis_name="c", subcore_axis_name="s")

    @functools.partial(
        pl.kernel,
        mesh=mesh,
        out_type=jax.ShapeDtypeStruct((HW, _CP), jnp.float32),
        scratch_types=dict(
            fx_v=pltpu.VMEM((span,), jnp.float32),
            fy_v=pltpu.VMEM((span,), jnp.float32),
            idx_v=pltpu.VMEM((2, 2, _CHUNK), jnp.int32),
            w_v=pltpu.VMEM((2, 4, _CHUNK), jnp.float32),
            rows_v=pltpu.VMEM((2, 2, _CHUNK, 2, _CP), jnp.float32),
            out_v=pltpu.VMEM((2, _CHUNK, _CP), jnp.float32),
            gsem0=pltpu.SemaphoreType.DMA,
            gsem1=pltpu.SemaphoreType.DMA,
            osem0=pltpu.SemaphoreType.DMA,
            osem1=pltpu.SemaphoreType.DMA,
        ),
    )
    def warp(t1_hbm, f_hbm, out_hbm, *, fx_v, fy_v, idx_v, w_v,
             rows_v, out_v, gsem0, gsem1, osem0, osem1):
        wid = lax.axis_index("s") * _NC + lax.axis_index("c")
        gsem = (gsem0, gsem1)
        osem = (osem0, osem1)

        def compute_chunk(ci, par):
            local0 = wid * span + ci * _CHUNK
            for g in range(_CHUNK // 16):
                lane = lax.iota(jnp.int32, 16)
                zero_i = jnp.zeros((16,), jnp.int32)
                one_i = jnp.full((16,), 1, jnp.int32)
                wvec = jnp.full((16,), W, jnp.int32)
                xmax = jnp.full((16,), W - 1, jnp.int32)
                ymax = jnp.full((16,), H - 1, jnp.int32)
                one_f = jnp.full((16,), 1.0, jnp.float32)
                sl = pl.ds(ci * _CHUNK + 16 * g, 16)
                q = local0 + 16 * g + lane
                y = lax.div(q, wvec)
                x = q - y * wvec
                x2 = x.astype(jnp.float32) + fx_v[sl]
                y2 = y.astype(jnp.float32) + fy_v[sl]
                ixL = jnp.minimum(jnp.maximum(x2.astype(jnp.int32), zero_i), xmax)
                iyT = jnp.minimum(jnp.maximum(y2.astype(jnp.int32), zero_i), ymax)
                ixR = jnp.minimum(ixL + one_i, xmax)
                iyB = jnp.minimum(iyT + one_i, ymax)
                alpha = x2 - ixL.astype(jnp.float32)
                beta = y2 - iyT.astype(jnp.float32)
                gsl = pl.ds(16 * g, 16)
                idx_v[par, 0, gsl] = iyT * wvec + ixL
                idx_v[par, 1, gsl] = iyB * wvec + ixL
                om_a = one_f - alpha
                om_b = one_f - beta
                w0 = om_a * om_b
                w1 = alpha * om_b
                w2 = om_a * beta
                w3 = alpha * beta
                d = (ixR - ixL).astype(jnp.float32)
                w1d = w1 * d
                w3d = w3 * d
                w_v[par, 0, gsl] = w0 + (w1 - w1d)
                w_v[par, 1, gsl] = w1d
                w_v[par, 2, gsl] = w2 + (w3 - w3d)
                w_v[par, 3, gsl] = w3d

        def fire_gathers(par):
            for k in range(2):
                pltpu.async_copy(t1_hbm.at[idx_v.at[par, k]],
                                 rows_v.at[par, k], gsem[par])

        def wait_gathers(par):
            for k in range(2):
                pltpu.make_async_copy(t1_hbm.at[idx_v.at[par, k]],
                                      rows_v.at[par, k], gsem[par]).wait()

        def blend_chunk(par):
            dnums = lax.GatherDimensionNumbers(
                offset_dims=(), collapsed_slice_dims=(0,),
                start_index_map=(0,))
            for g in range(_CHUNK // 16):
                gsl = pl.ds(16 * g, 16)

                def px_body(i, _, g=g, gsl=gsl):
                    zi = jnp.zeros((16,), jnp.int32)
                    i_splat = (i + zi)[:, None]
                    p = 16 * g + i
                    wv0 = w_v[par, 0, gsl]
                    wv1 = w_v[par, 1, gsl]
                    wv2 = w_v[par, 2, gsl]
                    wv3 = w_v[par, 3, gsl]
                    splat = lambda v: lax.gather(
                        v, i_splat, dnums, (1,),
                        mode=lax.GatherScatterMode.PROMISE_IN_BOUNDS)
                    w0 = splat(wv0)
                    w1 = splat(wv1)
                    w2 = splat(wv2)
                    w3 = splat(wv3)
                    for j in range(cblocks):
                        csl = pl.ds(16 * j, 16)
                        acc = w0 * rows_v[par, 0, p, 0, csl]
                        acc = acc + w1 * rows_v[par, 0, p, 1, csl]
                        acc = acc + w2 * rows_v[par, 1, p, 0, csl]
                        acc = acc + w3 * rows_v[par, 1, p, 1, csl]
                        out_v[par, p, csl] = acc
                    return 0

                lax.fori_loop(0, 16, px_body, 0)

        def out_slice(ci):
            return out_hbm.at[pl.ds(wid * span + ci * _CHUNK, _CHUNK)]

        pltpu.sync_copy(f_hbm.at[2 * b, pl.ds(wid * span, span)], fx_v)
        pltpu.sync_copy(f_hbm.at[2 * b + 1, pl.ds(wid * span, span)], fy_v)

        compute_chunk(jnp.int32(0), 0)
        fire_gathers(0)

        def pair_body(pair, _):
            for sub in range(2):
                c = 2 * pair + sub
                p_cur = sub
                p_nxt = 1 - sub

                @pl.when(c + 1 < n_chunks)
                def _():
                    compute_chunk(c + 1, p_nxt)
                    fire_gathers(p_nxt)

                wait_gathers(p_cur)
                blend_chunk(p_cur)

                @pl.when(c >= 2)
                def _():
                    pltpu.make_async_copy(out_v.at[p_cur],
                                          out_slice(c - 2),
                                          osem[p_cur]).wait()

                pltpu.async_copy(out_v.at[p_cur], out_slice(c), osem[p_cur])
            return 0

        lax.fori_loop(0, n_chunks // 2, pair_body, 0)
        for p in range(2):
            pltpu.make_async_copy(out_v.at[p],
                                  out_slice(n_chunks - 2 + p),
                                  osem[p]).wait()

    return warp


@jax.jit
def kernel(input1, input2):
    B, C, H, W = input1.shape
    flow = input2.reshape(B * 2, H * W)
    acc = None
    for b in range(B):
        table = _nchw_to_pairs(input1, b)
        rows = _make_warp(b, C, H, W)(table, flow)
        acc = _rows_to_nchw(rows, acc, b, B, C, H, W)
    return acc


# speedup vs baseline: 1.4488x; 1.4488x over previous
import functools

import jax
import jax.numpy as jnp
from jax import lax
from jax.experimental import pallas as pl
from jax.experimental.pallas import tpu as pltpu
from jax.experimental.pallas import tpu_sc as plsc

_NC = 2
_NS = 16
_NW = _NC * _NS

_CHUNK = 64
_CP = 128
_HB = 16


def _nchw_to_rows(x, b):
    B, C, H, W = x.shape

    def body(in_ref, out_ref):
        xb = in_ref[0]
        x2 = xb.reshape(C, _HB * W)
        out_ref[0, :, :C] = x2.T
        out_ref[0, :, C:] = jnp.zeros((_HB * W, _CP - C), jnp.float32)

    return pl.pallas_call(
        body,
        grid=(H // _HB,),
        in_specs=[pl.BlockSpec((1, C, _HB, W), lambda i, b=b: (b, 0, i, 0))],
        out_specs=pl.BlockSpec((1, _HB * W, _CP), lambda i: (0, i, 0)),
        out_shape=jax.ShapeDtypeStruct((1, H * W, _CP), jnp.float32),
    )(x)


def _rows_to_nchw(rows, acc, b, m, B, C, H, W):
    nblk = H // 2 // _HB

    def body(rows_ref, *refs):
        out_ref = refs[-1]
        t = rows_ref[0][:, :C].T
        out_ref[0] = t.reshape(C, _HB, W)

    in_specs = [pl.BlockSpec((1, _HB * W, _CP), lambda i: (0, i, 0))]
    args = [rows.reshape(1, H * W // 2, _CP)]
    aliases = {}
    if acc is not None:
        in_specs.append(pl.BlockSpec(memory_space=pl.ANY))
        args.append(acc)
        aliases = {1: 0}

    return pl.pallas_call(
        body,
        grid=(nblk,),
        in_specs=in_specs,
        out_specs=pl.BlockSpec((1, C, _HB, W),
                               lambda i, b=b, m=m: (b, 0, m * nblk + i, 0)),
        out_shape=jax.ShapeDtypeStruct((B, C, H, W), jnp.float32),
        input_output_aliases=aliases,
    )(*args)


def _make_warp(b, m, C, H, W):
    HW = H * W
    HW2 = HW // 2
    assert HW2 % _NW == 0
    span = HW2 // _NW
    n_chunks = span // _CHUNK
    assert span % _CHUNK == 0 and n_chunks % 2 == 0
    cblocks = C // 16
    assert C % 16 == 0

    mesh = plsc.VectorSubcoreMesh(core_axis_name="c", subcore_axis_name="s")

    @functools.partial(
        pl.kernel,
        mesh=mesh,
        out_type=jax.ShapeDtypeStruct((HW2, _CP), jnp.float32),
        scratch_types=dict(
            fx_v=pltpu.VMEM((span,), jnp.float32),
            fy_v=pltpu.VMEM((span,), jnp.float32),
            idx_v=pltpu.VMEM((2, 4, _CHUNK), jnp.int32),
            w_v=pltpu.VMEM((2, 4, _CHUNK), jnp.float32),
            rows_v=pltpu.VMEM((2, 4, _CHUNK, _CP), jnp.float32),
            out_v=pltpu.VMEM((2, _CHUNK, _CP), jnp.float32),
            gsem0=pltpu.SemaphoreType.DMA,
            gsem1=pltpu.SemaphoreType.DMA,
            osem0=pltpu.SemaphoreType.DMA,
            osem1=pltpu.SemaphoreType.DMA,
        ),
    )
    def warp(t1_hbm, f_hbm, out_hbm, *, fx_v, fy_v, idx_v, w_v,
             rows_v, out_v, gsem0, gsem1, osem0, osem1):
        wid = lax.axis_index("s") * _NC + lax.axis_index("c")
        gsem = (gsem0, gsem1)
        osem = (osem0, osem1)

        def compute_chunk(ci, par):
            local0 = m * HW2 + wid * span + ci * _CHUNK
            for g in range(_CHUNK // 16):
                lane = lax.iota(jnp.int32, 16)
                zero_i = jnp.zeros((16,), jnp.int32)
                one_i = jnp.full((16,), 1, jnp.int32)
                wvec = jnp.full((16,), W, jnp.int32)
                xmax = jnp.full((16,), W - 1, jnp.int32)
                ymax = jnp.full((16,), H - 1, jnp.int32)
                one_f = jnp.full((16,), 1.0, jnp.float32)
                sl = pl.ds(ci * _CHUNK + 16 * g, 16)
                q = local0 + 16 * g + lane
                y = lax.div(q, wvec)
                x = q - y * wvec
                x2 = x.astype(jnp.float32) + fx_v[sl]
                y2 = y.astype(jnp.float32) + fy_v[sl]
                ixL = jnp.minimum(jnp.maximum(x2.astype(jnp.int32), zero_i), xmax)
                iyT = jnp.minimum(jnp.maximum(y2.astype(jnp.int32), zero_i), ymax)
                ixR = jnp.minimum(ixL + one_i, xmax)
                iyB = jnp.minimum(iyT + one_i, ymax)
                alpha = x2 - ixL.astype(jnp.float32)
                beta = y2 - iyT.astype(jnp.float32)
                gsl = pl.ds(16 * g, 16)
                idx_v[par, 0, gsl] = iyT * wvec + ixL
                idx_v[par, 1, gsl] = iyT * wvec + ixR
                idx_v[par, 2, gsl] = iyB * wvec + ixL
                idx_v[par, 3, gsl] = iyB * wvec + ixR
                om_a = one_f - alpha
                om_b = one_f - beta
                w_v[par, 0, gsl] = om_a * om_b
                w_v[par, 1, gsl] = alpha * om_b
                w_v[par, 2, gsl] = om_a * beta
                w_v[par, 3, gsl] = alpha * beta

        def fire_gathers(par):
            for k in range(4):
                pltpu.async_copy(t1_hbm.at[idx_v.at[par, k]],
                                 rows_v.at[par, k], gsem[par])

        def wait_gathers(par):
            for k in range(4):
                pltpu.make_async_copy(t1_hbm.at[idx_v.at[par, k]],
                                      rows_v.at[par, k], gsem[par]).wait()

        def blend_chunk(par):
            dnums = lax.GatherDimensionNumbers(
                offset_dims=(), collapsed_slice_dims=(0,),
                start_index_map=(0,))
            for g in range(_CHUNK // 16):
                gsl = pl.ds(16 * g, 16)

                def px_body(i, _, g=g, gsl=gsl):
                    zi = jnp.zeros((16,), jnp.int32)
                    i_splat = (i + zi)[:, None]
                    p = 16 * g + i
                    wv0 = w_v[par, 0, gsl]
                    wv1 = w_v[par, 1, gsl]
                    wv2 = w_v[par, 2, gsl]
                    wv3 = w_v[par, 3, gsl]
                    splat = lambda v: lax.gather(
                        v, i_splat, dnums, (1,),
                        mode=lax.GatherScatterMode.PROMISE_IN_BOUNDS)
                    w0 = splat(wv0)
                    w1 = splat(wv1)
                    w2 = splat(wv2)
                    w3 = splat(wv3)
                    for j in range(cblocks):
                        csl = pl.ds(16 * j, 16)
                        acc = w0 * rows_v[par, 0, p, csl]
                        acc = acc + w1 * rows_v[par, 1, p, csl]
                        acc = acc + w2 * rows_v[par, 2, p, csl]
                        acc = acc + w3 * rows_v[par, 3, p, csl]
                        out_v[par, p, csl] = acc
                    return 0

                lax.fori_loop(0, 16, px_body, 0)

        def out_slice(ci):
            return out_hbm.at[pl.ds(wid * span + ci * _CHUNK, _CHUNK)]

        fbase = m * HW2 + wid * span
        pltpu.sync_copy(f_hbm.at[2 * b, pl.ds(fbase, span)], fx_v)
        pltpu.sync_copy(f_hbm.at[2 * b + 1, pl.ds(fbase, span)], fy_v)

        compute_chunk(jnp.int32(0), 0)
        fire_gathers(0)

        def pair_body(pair, _):
            for sub in range(2):
                c = 2 * pair + sub
                p_cur = sub
                p_nxt = 1 - sub

                @pl.when(c + 1 < n_chunks)
                def _():
                    compute_chunk(c + 1, p_nxt)
                    fire_gathers(p_nxt)

                wait_gathers(p_cur)
                blend_chunk(p_cur)

                @pl.when(c >= 2)
                def _():
                    pltpu.make_async_copy(out_v.at[p_cur],
                                          out_slice(c - 2),
                                          osem[p_cur]).wait()

                pltpu.async_copy(out_v.at[p_cur], out_slice(c), osem[p_cur])
            return 0

        lax.fori_loop(0, n_chunks // 2, pair_body, 0)
        for p in range(2):
            pltpu.make_async_copy(out_v.at[p],
                                  out_slice(n_chunks - 2 + p),
                                  osem[p]).wait()

    return warp


@jax.jit
def kernel(input1, input2):
    B, C, H, W = input1.shape
    flow = input2.reshape(B * 2, H * W)
    acc = None
    for b in range(B):
        table = _nchw_to_rows(input1, b).reshape(H * W, _CP)
        for m in range(2):
            rows = _make_warp(b, m, C, H, W)(table, flow)
            acc = _rows_to_nchw(rows, acc, b, m, B, C, H, W)
    return acc


# speedup vs baseline: 1.4872x; 1.0265x over previous
import functools

import jax
import jax.numpy as jnp
from jax import lax
from jax.experimental import pallas as pl
from jax.experimental.pallas import tpu as pltpu
from jax.experimental.pallas import tpu_sc as plsc

_NC = 2
_NS = 16
_NW = _NC * _NS

_CHUNK = 64
_CP = 128
_HB = 32


def _nchw_to_rows(x, b):
    B, C, H, W = x.shape

    def body(in_ref, out_ref):
        xb = in_ref[0]
        x2 = xb.reshape(C, _HB * W)
        out_ref[0, :, :C] = x2.T
        out_ref[0, :, C:] = jnp.zeros((_HB * W, _CP - C), jnp.float32)

    return pl.pallas_call(
        body,
        grid=(H // _HB,),
        in_specs=[pl.BlockSpec((1, C, _HB, W), lambda i, b=b: (b, 0, i, 0))],
        out_specs=pl.BlockSpec((1, _HB * W, _CP), lambda i: (0, i, 0)),
        out_shape=jax.ShapeDtypeStruct((1, H * W, _CP), jnp.float32),
    )(x)


def _rows_to_nchw(rows, acc, b, B, C, H, W):
    def body(rows_ref, *refs):
        out_ref = refs[-1]
        t = rows_ref[0][:, :C].T
        out_ref[0] = t.reshape(C, _HB, W)

    in_specs = [pl.BlockSpec((1, _HB * W, _CP), lambda i: (0, i, 0))]
    args = [rows.reshape(1, H * W, _CP)]
    aliases = {}
    if acc is not None:
        in_specs.append(pl.BlockSpec(memory_space=pl.ANY))
        args.append(acc)
        aliases = {1: 0}

    return pl.pallas_call(
        body,
        grid=(H // _HB,),
        in_specs=in_specs,
        out_specs=pl.BlockSpec((1, C, _HB, W), lambda i, b=b: (b, 0, i, 0)),
        out_shape=jax.ShapeDtypeStruct((B, C, H, W), jnp.float32),
        input_output_aliases=aliases,
    )(*args)


def _make_warp(b, C, H, W):
    HW = H * W
    assert HW % _NW == 0
    span = HW // _NW
    n_chunks = span // _CHUNK
    assert span % _CHUNK == 0 and n_chunks % 2 == 0
    cblocks = C // 16
    assert C % 16 == 0

    mesh = plsc.VectorSubcoreMesh(core_axis_name="c", subcore_axis_name="s")

    @functools.partial(
        pl.kernel,
        mesh=mesh,
        out_type=jax.ShapeDtypeStruct((HW, _CP), jnp.float32),
        scratch_types=dict(
            fx_v=pltpu.VMEM((span,), jnp.float32),
            fy_v=pltpu.VMEM((span,), jnp.float32),
            idx_v=pltpu.VMEM((2, 4, _CHUNK), jnp.int32),
            w_v=pltpu.VMEM((2, 4, _CHUNK), jnp.float32),
            rows_v=pltpu.VMEM((2, 4, _CHUNK, _CP), jnp.float32),
            out_v=pltpu.VMEM((2, _CHUNK, _CP), jnp.float32),
            gsem0=pltpu.SemaphoreType.DMA,
            gsem1=pltpu.SemaphoreType.DMA,
            osem0=pltpu.SemaphoreType.DMA,
            osem1=pltpu.SemaphoreType.DMA,
        ),
    )
    def warp(t1_hbm, f_hbm, out_hbm, *, fx_v, fy_v, idx_v, w_v,
             rows_v, out_v, gsem0, gsem1, osem0, osem1):
        wid = lax.axis_index("s") * _NC + lax.axis_index("c")
        gsem = (gsem0, gsem1)
        osem = (osem0, osem1)

        def compute_chunk(ci, par):
            local0 = wid * span + ci * _CHUNK
            for g in range(_CHUNK // 16):
                lane = lax.iota(jnp.int32, 16)
                zero_i = jnp.zeros((16,), jnp.int32)
                one_i = jnp.full((16,), 1, jnp.int32)
                wvec = jnp.full((16,), W, jnp.int32)
                xmax = jnp.full((16,), W - 1, jnp.int32)
                ymax = jnp.full((16,), H - 1, jnp.int32)
                one_f = jnp.full((16,), 1.0, jnp.float32)
                sl = pl.ds(ci * _CHUNK + 16 * g, 16)
                q = local0 + 16 * g + lane
                y = lax.div(q, wvec)
                x = q - y * wvec
                x2 = x.astype(jnp.float32) + fx_v[sl]
                y2 = y.astype(jnp.float32) + fy_v[sl]
                ixL = jnp.minimum(jnp.maximum(x2.astype(jnp.int32), zero_i), xmax)
                iyT = jnp.minimum(jnp.maximum(y2.astype(jnp.int32), zero_i), ymax)
                ixR = jnp.minimum(ixL + one_i, xmax)
                iyB = jnp.minimum(iyT + one_i, ymax)
                alpha = x2 - ixL.astype(jnp.float32)
                beta = y2 - iyT.astype(jnp.float32)
                gsl = pl.ds(16 * g, 16)
                idx_v[par, 0, gsl] = iyT * wvec + ixL
                idx_v[par, 1, gsl] = iyT * wvec + ixR
                idx_v[par, 2, gsl] = iyB * wvec + ixL
                idx_v[par, 3, gsl] = iyB * wvec + ixR
                om_a = one_f - alpha
                om_b = one_f - beta
                w_v[par, 0, gsl] = om_a * om_b
                w_v[par, 1, gsl] = alpha * om_b
                w_v[par, 2, gsl] = om_a * beta
                w_v[par, 3, gsl] = alpha * beta

        def fire_gathers(par):
            for k in range(4):
                pltpu.async_copy(t1_hbm.at[idx_v.at[par, k]],
                                 rows_v.at[par, k], gsem[par])

        def wait_gathers(par):
            for k in range(4):
                pltpu.make_async_copy(t1_hbm.at[idx_v.at[par, k]],
                                      rows_v.at[par, k], gsem[par]).wait()

        def blend_chunk(par):
            dnums = lax.GatherDimensionNumbers(
                offset_dims=(), collapsed_slice_dims=(0,),
                start_index_map=(0,))
            for g in range(_CHUNK // 16):
                gsl = pl.ds(16 * g, 16)

                def px_body(i, _, g=g, gsl=gsl):
                    zi = jnp.zeros((16,), jnp.int32)
                    i_splat = (i + zi)[:, None]
                    p = 16 * g + i
                    wv0 = w_v[par, 0, gsl]
                    wv1 = w_v[par, 1, gsl]
                    wv2 = w_v[par, 2, gsl]
                    wv3 = w_v[par, 3, gsl]
                    splat = lambda v: lax.gather(
                        v, i_splat, dnums, (1,),
                        mode=lax.GatherScatterMode.PROMISE_IN_BOUNDS)
                    w0 = splat(wv0)
                    w1 = splat(wv1)
                    w2 = splat(wv2)
                    w3 = splat(wv3)
                    for j in range(cblocks):
                        csl = pl.ds(16 * j, 16)
                        acc = w0 * rows_v[par, 0, p, csl]
                        acc = acc + w1 * rows_v[par, 1, p, csl]
                        acc = acc + w2 * rows_v[par, 2, p, csl]
                        acc = acc + w3 * rows_v[par, 3, p, csl]
                        out_v[par, p, csl] = acc
                    return 0

                lax.fori_loop(0, 16, px_body, 0)

        def out_slice(ci):
            return out_hbm.at[pl.ds(wid * span + ci * _CHUNK, _CHUNK)]

        pltpu.sync_copy(f_hbm.at[2 * b, pl.ds(wid * span, span)], fx_v)
        pltpu.sync_copy(f_hbm.at[2 * b + 1, pl.ds(wid * span, span)], fy_v)

        compute_chunk(jnp.int32(0), 0)
        fire_gathers(0)

        def pair_body(pair, _):
            for sub in range(2):
                c = 2 * pair + sub
                p_cur = sub
                p_nxt = 1 - sub

                @pl.when(c + 1 < n_chunks)
                def _():
                    compute_chunk(c + 1, p_nxt)
                    fire_gathers(p_nxt)

                wait_gathers(p_cur)
                blend_chunk(p_cur)

                @pl.when(c >= 2)
                def _():
                    pltpu.make_async_copy(out_v.at[p_cur],
                                          out_slice(c - 2),
                                          osem[p_cur]).wait()

                pltpu.async_copy(out_v.at[p_cur], out_slice(c), osem[p_cur])
            return 0

        lax.fori_loop(0, n_chunks // 2, pair_body, 0)
        for p in range(2):
            pltpu.make_async_copy(out_v.at[p],
                                  out_slice(n_chunks - 2 + p),
                                  osem[p]).wait()

    return warp


@jax.jit
def kernel(input1, input2):
    B, C, H, W = input1.shape
    flow = input2.reshape(B * 2, H * W)
    acc = None
    for b in range(B):
        table = _nchw_to_rows(input1, b).reshape(H * W, _CP)
        rows = _make_warp(b, C, H, W)(table, flow)
        acc = _rows_to_nchw(rows, acc, b, B, C, H, W)
    return acc
